# Initial kernel scaffold; baseline (speedup 1.0000x reference)
import jax
import jax.numpy as jnp
from jax.experimental import pallas as pl


def kernel(x):
    raise NotImplementedError("write your pallas kernel here")


# speedup vs baseline: 1.3109x; 1.3109x over previous
import numpy as np
import jax
import jax.numpy as jnp
from jax.experimental import pallas as pl
from jax.experimental.pallas import tpu as pltpu

_B, _S, _F = 4096, 50, 100
_N = _B * _S * _F
_LANES = 128
_ROWS = _N // _LANES
_RB = 1000

_THR = np.uint32(2516583 << 9)

_KS = (np.uint32(0), np.uint32(1), np.uint32(0x1BD11BDB))
_ROTS = ((13, 15, 26, 6), (17, 29, 16, 24))


def _rotl(v, r):
    return (v << np.uint32(r)) | (v >> np.uint32(32 - r))


def _threefry_bits(j):
    x0 = jnp.zeros_like(j) + _KS[0]
    x1 = j + _KS[1]
    for i in range(5):
        for r in _ROTS[i % 2]:
            x0 = x0 + x1
            x1 = _rotl(x1, r)
            x1 = x1 ^ x0
        x0 = x0 + _KS[(i + 1) % 3]
        x1 = x1 + np.uint32((int(_KS[(i + 2) % 3]) + i + 1) & 0xFFFFFFFF)
    return x0 ^ x1


def _body(x_ref, fv_ref, out_ref, mask_ref):
    i = pl.program_id(0)
    base = jnp.uint32(i) * jnp.uint32(_RB * _LANES)
    rr = jax.lax.broadcasted_iota(jnp.int32, (_RB, _LANES), 0).astype(jnp.uint32)
    ll = jax.lax.broadcasted_iota(jnp.int32, (_RB, _LANES), 1).astype(jnp.uint32)
    j = base + rr * jnp.uint32(_LANES) + ll
    m = _threefry_bits(j) < _THR
    out_ref[...] = jnp.where(m, fv_ref[...], x_ref[...])
    mask_ref[...] = m.astype(jnp.float32)


def _featvals():
    k2 = jax.random.key(2)
    k3 = jax.random.key(3)
    vals = [
        jax.random.randint(jax.random.fold_in(k2, f), (), 0, 10).astype(jnp.float32)
        for f in range(26)
    ]
    vals += [
        jax.random.normal(jax.random.fold_in(k3, f), (), dtype=jnp.float32)
        for f in range(26, 100)
    ]
    return jnp.stack(vals)


_FV_IDX = (np.arange(_RB * _LANES) % _F).reshape(_RB, _LANES)


def kernel(x):
    xr = x.reshape(_ROWS, _LANES)
    fv = _featvals()[_FV_IDX]
    out, mask = pl.pallas_call(
        _body,
        grid=(_ROWS // _RB,),
        in_specs=[
            pl.BlockSpec((_RB, _LANES), lambda i: (i, 0)),
            pl.BlockSpec((_RB, _LANES), lambda i: (0, 0)),
        ],
        out_specs=[
            pl.BlockSpec((_RB, _LANES), lambda i: (i, 0)),
            pl.BlockSpec((_RB, _LANES), lambda i: (i, 0)),
        ],
        out_shape=[jax.ShapeDtypeStruct((_ROWS, _LANES), jnp.float32)] * 2,
        compiler_params=pltpu.CompilerParams(dimension_semantics=("arbitrary",)),
    )(xr, fv)
    return out.reshape(_B, _S, _F), mask.reshape(_B, _S, _F)
